# Initial kernel scaffold; baseline (speedup 1.0000x reference)
import functools

import jax
import jax.numpy as jnp
from jax import lax
from jax.experimental import pallas as pl
from jax.experimental.pallas import tpu as pltpu
from jax.experimental.pallas import tpu_sc as plsc

N = 10000
NPAD = 10240
E = 320000
FEAT = 128
IDXN = 1024

NC = 2
NS = 16
NW = NC * NS
ET = E // NW
CB = 80
NCH = ET // CB
ROWS_T = NPAD // NS
SLT = NPAD // NW
GIDX_T = IDXN // NS

_mesh = lambda: plsc.VectorSubcoreMesh(core_axis_name="c", subcore_axis_name="s")


def _zero_vec():
  return jnp.zeros((16,), jnp.float32)


def _iota16():
  return lax.broadcasted_iota(jnp.int32, (16,), 0)


def _deg_kernel(dst_hbm, ew_hbm, out_hbm, deg_priv, dstv, ewv, slots_sh, red_v, red_o):
  c = lax.axis_index("c")
  s = lax.axis_index("s")
  wid = c * NS + s
  base_e = wid * ET

  def zbody(i, _):
    deg_priv[pl.ds(i * 16, 16)] = _zero_vec()
    return 0
  lax.fori_loop(0, NPAD // 16, zbody, 0)

  CE = 2000

  def chunk(k, _):
    off = base_e + k * CE
    pltpu.sync_copy(dst_hbm.at[pl.ds(off, CE)], dstv)
    pltpu.sync_copy(ew_hbm.at[pl.ds(off, CE)], ewv)

    def body(j, _):
      idx = dstv[pl.ds(j * 16, 16)]
      vals = ewv[pl.ds(j * 16, 16)]
      plsc.addupdate_scatter(deg_priv, [idx], vals)
      return 0
    lax.fori_loop(0, CE // 16, body, 0)
    return 0
  lax.fori_loop(0, ET // CE, chunk, 0)

  pltpu.sync_copy(deg_priv, slots_sh.at[s])
  plsc.subcore_barrier()
  part0 = s * (NPAD // NS)
  for t in range(NS):
    pltpu.sync_copy(slots_sh.at[t, pl.ds(part0, NPAD // NS)], red_v.at[t])

  def rbody(i, _):
    acc = red_v[0, pl.ds(i * 16, 16)]
    for t in range(1, NS):
      acc = acc + red_v[t, pl.ds(i * 16, 16)]
    red_o[pl.ds(i * 16, 16)] = acc
    return 0
  lax.fori_loop(0, (NPAD // NS) // 16, rbody, 0)
  pltpu.sync_copy(red_o, out_hbm.at[c, pl.ds(part0, NPAD // NS)])


def _run_deg(dst, ew):
  k = pl.kernel(
      _deg_kernel,
      out_type=jax.ShapeDtypeStruct((NC, NPAD), jnp.float32),
      mesh=_mesh(),
      scratch_types=[
          pltpu.VMEM((NPAD,), jnp.float32),
          pltpu.VMEM((2000,), jnp.int32),
          pltpu.VMEM((2000,), jnp.float32),
          pltpu.VMEM_SHARED((NS, NPAD), jnp.float32),
          pltpu.VMEM((NS, NPAD // NS), jnp.float32),
          pltpu.VMEM((NPAD // NS,), jnp.float32),
      ],
  )
  return k(dst, ew)


def _scatter_body(gather_idx, g_hbm, dinv_hbm, src_hbm, dst_hbm, ew_hbm, idx_hbm,
                  acc_out, rows_out, acc_sh, dinv_v, srcv, dstv, ewv, fv, rows_v,
                  zero_v, ids_v, idxg_v, grows_v):
  c = lax.axis_index("c")
  s = lax.axis_index("s")
  wid = c * NS + s
  base_e = wid * ET
  r0 = s * ROWS_T

  pltpu.sync_copy(dinv_hbm, dinv_v)

  def zb(i, _):
    for j in range(8):
      zero_v[i, pl.ds(j * 16, 16)] = _zero_vec()
    return 0
  lax.fori_loop(0, CB, zb, 0)
  for kk in range(ROWS_T // CB):
    pltpu.sync_copy(zero_v, acc_sh.at[pl.ds(r0 + kk * CB, CB)])
  plsc.subcore_barrier()

  def scale_rows(nrows):
    def sb(e, _):
      sv = fv[e]
      row = rows_v.at[e]
      for j in range(8):
        row[pl.ds(j * 16, 16)] = row[pl.ds(j * 16, 16)] * sv
      return 0
    lax.fori_loop(0, nrows, sb, 0)

  def chunk(k, _):
    off = base_e + k * CB
    pltpu.sync_copy(src_hbm.at[pl.ds(off, CB)], srcv)
    pltpu.sync_copy(dst_hbm.at[pl.ds(off, CB)], dstv)
    pltpu.sync_copy(ew_hbm.at[pl.ds(off, CB)], ewv)
    for j in range(CB // 16):
      sl = pl.ds(j * 16, 16)
      ds_ = plsc.load_gather(dinv_v, [srcv[sl]])
      dd_ = plsc.load_gather(dinv_v, [dstv[sl]])
      fv[sl] = ewv[sl] * ds_ * dd_
    pltpu.sync_copy(g_hbm.at[srcv], rows_v)
    scale_rows(CB)
    pltpu.sync_copy(rows_v, acc_sh.at[dstv], add=True)
    return 0
  lax.fori_loop(0, NCH, chunk, 0)

  sl_base = wid * SLT

  def slchunk(k, _):
    base = sl_base + k * CB
    for j in range(CB // 16):
      sl = pl.ds(j * 16, 16)
      ids_v[sl] = _iota16() + (base + j * 16)
      dv = dinv_v[pl.ds(base + j * 16, 16)]
      fv[sl] = dv * dv
    pltpu.sync_copy(g_hbm.at[pl.ds(base, CB)], rows_v)
    scale_rows(CB)
    pltpu.sync_copy(rows_v, acc_sh.at[ids_v], add=True)
    return 0
  lax.fori_loop(0, SLT // CB, slchunk, 0)

  plsc.subcore_barrier()
  if gather_idx:
    pltpu.sync_copy(idx_hbm.at[pl.ds(s * GIDX_T, GIDX_T)], idxg_v)
    pltpu.sync_copy(acc_sh.at[idxg_v], grows_v)
    pltpu.sync_copy(grows_v, rows_out.at[c, pl.ds(s * GIDX_T, GIDX_T)])
  else:
    pltpu.sync_copy(acc_sh.at[pl.ds(r0, ROWS_T)], acc_out.at[c, pl.ds(r0, ROWS_T)])


def _run_scatter(g, dinv, src, dst, ew, idx, gather_idx):
  scratch = [
      pltpu.VMEM_SHARED((NPAD, FEAT), jnp.float32),
      pltpu.VMEM((NPAD,), jnp.float32),
      pltpu.VMEM((CB,), jnp.int32),
      pltpu.VMEM((CB,), jnp.int32),
      pltpu.VMEM((CB,), jnp.float32),
      pltpu.VMEM((CB,), jnp.float32),
      pltpu.VMEM((CB, FEAT), jnp.float32),
      pltpu.VMEM((CB, FEAT), jnp.float32),
      pltpu.VMEM((CB,), jnp.int32),
      pltpu.VMEM((GIDX_T,), jnp.int32),
      pltpu.VMEM((GIDX_T, FEAT), jnp.float32),
  ]
  if gather_idx:
    out_type = [jax.ShapeDtypeStruct((NC, IDXN, FEAT), jnp.float32)]
    def body(g_, dinv_, src_, dst_, ew_, idx_, rows_out, *scr):
      _scatter_body(True, g_, dinv_, src_, dst_, ew_, idx_, None, rows_out, *scr)
  else:
    out_type = [jax.ShapeDtypeStruct((NC, NPAD, FEAT), jnp.float32)]
    def body(g_, dinv_, src_, dst_, ew_, idx_, acc_out, *scr):
      _scatter_body(False, g_, dinv_, src_, dst_, ew_, idx_, acc_out, None, *scr)
  k = pl.kernel(body, out_type=out_type, mesh=_mesh(), scratch_types=scratch)
  return k(g, dinv, src, dst, ew, idx)[0]


def _tc1_body(vf, tf, degp, fc1wT, fc1b, fc2wT, fc2b, wAT, wBT, rb, c0wT,
              g0_ref, dinv_ref):
  f32 = jnp.float32
  v = jnp.dot(vf[...], fc1wT[...], preferred_element_type=f32) + fc1b[...]
  t = jnp.dot(tf[...], fc2wT[...], preferred_element_type=f32) + fc2b[...]
  h = (jnp.dot(v, wAT[...], preferred_element_type=f32)
       + jnp.dot(t, wBT[...], preferred_element_type=f32) + rb[...])
  h = jnp.where(h >= 0, h, 0.01 * h)
  g0_ref[...] = jnp.dot(h, c0wT[...], preferred_element_type=f32)
  deg = 1.0 + degp[0] + degp[1]
  dinv_ref[...] = lax.rsqrt(deg)


def _run_tc1(vf, tf, degp, fc1wT, fc1b, fc2wT, fc2b, wAT, wBT, rb, c0wT):
  BR = 640
  grid = (NPAD // BR,)
  full = lambda shape: pl.BlockSpec(shape, lambda i: tuple(0 for _ in shape))
  return pl.pallas_call(
      _tc1_body,
      grid=grid,
      in_specs=[
          pl.BlockSpec((BR, 16), lambda i: (i, 0)),
          pl.BlockSpec((BR, 768), lambda i: (i, 0)),
          full((NC, 80, 128)),
          full((16, 128)), full((1, 128)),
          full((768, 128)), full((1, 128)),
          full((128, 128)), full((128, 128)), full((1, 128)),
          full((128, 128)),
      ],
      out_specs=[
          pl.BlockSpec((BR, 128), lambda i: (i, 0)),
          pl.BlockSpec((80, 128), lambda i: (0, 0)),
      ],
      out_shape=[
          jax.ShapeDtypeStruct((NPAD, 128), jnp.float32),
          jax.ShapeDtypeStruct((80, 128), jnp.float32),
      ],
  )(vf, tf, degp, fc1wT, fc1b, fc2wT, fc2b, wAT, wBT, rb, c0wT)


def _tc2_body(acc, b0, c1wT, g1_ref):
  x = acc[0] + acc[1] + b0[...]
  g1_ref[...] = jnp.dot(x, c1wT[...], preferred_element_type=jnp.float32)


def _run_tc2(acc, b0, c1wT):
  BR = 640
  return pl.pallas_call(
      _tc2_body,
      grid=(NPAD // BR,),
      in_specs=[
          pl.BlockSpec((NC, BR, 128), lambda i: (0, i, 0)),
          pl.BlockSpec((1, 128), lambda i: (0, 0)),
          pl.BlockSpec((128, 128), lambda i: (0, 0)),
      ],
      out_specs=pl.BlockSpec((BR, 128), lambda i: (i, 0)),
      out_shape=jax.ShapeDtypeStruct((NPAD, 128), jnp.float32),
  )(acc, b0, c1wT)


def _tc3_body(rows, b1, fc3pT, fc3bp, out_ref):
  x = rows[0] + rows[1] + b1[...]
  out_ref[...] = jnp.dot(x, fc3pT[...], preferred_element_type=jnp.float32) + fc3bp[...]


def _run_tc3(rows, b1, fc3pT, fc3bp):
  return pl.pallas_call(
      _tc3_body,
      grid=(1,),
      in_specs=[
          pl.BlockSpec((NC, IDXN, 128), lambda i: (0, 0, 0)),
          pl.BlockSpec((1, 128), lambda i: (0, 0)),
          pl.BlockSpec((128, 128), lambda i: (0, 0)),
          pl.BlockSpec((1, 128), lambda i: (0, 0)),
      ],
      out_specs=pl.BlockSpec((IDXN, 128), lambda i: (0, 0)),
      out_shape=jax.ShapeDtypeStruct((IDXN, 128), jnp.float32),
  )(rows, b1, fc3pT, fc3bp)


def kernel(value_feature, text_feature, edge_index, edge_weight, idx,
           fc1_w, fc1_b, fc2_w, fc2_b, relu_w, relu_b,
           conv0_w, conv0_b, conv1_w, conv1_b, fc3_w, fc3_b):
  f32 = jnp.float32
  vf = jnp.zeros((NPAD, 16), f32).at[:N].set(value_feature)
  tf = jnp.zeros((NPAD, 768), f32).at[:N].set(text_feature)
  src = edge_index[0]
  dst = edge_index[1]

  fc1wT = fc1_w.T
  fc2wT = fc2_w.T
  wAT = relu_w[:, :128].T
  wBT = relu_w[:, 128:].T
  c0wT = conv0_w.T
  c1wT = conv1_w.T
  fc3pT = jnp.zeros((128, 128), f32).at[:, :2].set(fc3_w.T)
  fc3bp = jnp.zeros((1, 128), f32).at[0, :2].set(fc3_b)
  r1 = lambda b: b.reshape(1, 128)

  degp = _run_deg(dst, edge_weight)
  g0, dinv2d = _run_tc1(vf, tf, degp.reshape(NC, 80, 128), fc1wT, r1(fc1_b),
                        fc2wT, r1(fc2_b), wAT, wBT, r1(relu_b), c0wT)
  dinv = dinv2d.reshape(NPAD)
  acc0 = _run_scatter(g0, dinv, src, dst, edge_weight, idx, gather_idx=False)
  g1 = _run_tc2(acc0, r1(conv0_b), c1wT)
  rows = _run_scatter(g1, dinv, src, dst, edge_weight, idx, gather_idx=True)
  out = _run_tc3(rows, r1(conv1_b), fc3pT, fc3bp)
  return out[:, :2]


# speedup vs baseline: 9.5913x; 9.5913x over previous
import functools

import jax
import jax.numpy as jnp
from jax import lax
from jax.experimental import pallas as pl
from jax.experimental.pallas import tpu as pltpu
from jax.experimental.pallas import tpu_sc as plsc

N = 10000
NPAD = 10240
E = 320000
FEAT = 128
IDXN = 1024

NC = 2
NS = 16
NW = NC * NS
ET = E // NW
CB = 80
NCH = ET // CB
ROWS_T = NPAD // NS
SLT = NPAD // NW
GIDX_T = IDXN // NS

_mesh = lambda: plsc.VectorSubcoreMesh(core_axis_name="c", subcore_axis_name="s")


def _zero_vec():
  return jnp.zeros((16,), jnp.float32)


def _iota16():
  return lax.broadcasted_iota(jnp.int32, (16,), 0)


def _deg_kernel(dst_hbm, ew_hbm, out_hbm, deg_priv, dstv, ewv, slots_sh, red_v, red_o):
  c = lax.axis_index("c")
  s = lax.axis_index("s")
  wid = c * NS + s
  base_e = wid * ET

  def zbody(i, _):
    deg_priv[pl.ds(i * 16, 16)] = _zero_vec()
    return 0
  lax.fori_loop(0, NPAD // 16, zbody, 0)

  CE = 2000

  def chunk(k, _):
    off = base_e + k * CE
    pltpu.sync_copy(dst_hbm.at[pl.ds(off, CE)], dstv)
    pltpu.sync_copy(ew_hbm.at[pl.ds(off, CE)], ewv)

    def body(j, _):
      idx = dstv[pl.ds(j * 16, 16)]
      vals = ewv[pl.ds(j * 16, 16)]
      plsc.addupdate_scatter(deg_priv, [idx], vals)
      return 0
    lax.fori_loop(0, CE // 16, body, 0)
    return 0
  lax.fori_loop(0, ET // CE, chunk, 0)

  pltpu.sync_copy(deg_priv, slots_sh.at[s])
  plsc.subcore_barrier()
  part0 = s * (NPAD // NS)
  for t in range(NS):
    pltpu.sync_copy(slots_sh.at[t, pl.ds(part0, NPAD // NS)], red_v.at[t])

  def rbody(i, _):
    acc = red_v[0, pl.ds(i * 16, 16)]
    for t in range(1, NS):
      acc = acc + red_v[t, pl.ds(i * 16, 16)]
    red_o[pl.ds(i * 16, 16)] = acc
    return 0
  lax.fori_loop(0, (NPAD // NS) // 16, rbody, 0)
  pltpu.sync_copy(red_o, out_hbm.at[c, pl.ds(part0, NPAD // NS)])


def _run_deg(dst, ew):
  k = pl.kernel(
      _deg_kernel,
      out_type=jax.ShapeDtypeStruct((NC, NPAD), jnp.float32),
      mesh=_mesh(),
      compiler_params=pltpu.CompilerParams(needs_layout_passes=False),
      scratch_types=[
          pltpu.VMEM((NPAD,), jnp.float32),
          pltpu.VMEM((2000,), jnp.int32),
          pltpu.VMEM((2000,), jnp.float32),
          pltpu.VMEM_SHARED((NS, NPAD), jnp.float32),
          pltpu.VMEM((NS, NPAD // NS), jnp.float32),
          pltpu.VMEM((NPAD // NS,), jnp.float32),
      ],
  )
  return k(dst, ew)


def _scatter_body(gather_idx, g_hbm, dinv_hbm, src_hbm, dst_hbm, ew_hbm, idx_hbm,
                  acc_out, rows_out, acc_sh, dinv_v, srcv, dstv, ewv, fv, rows_v,
                  zero_v, ids_v, idxg_v, grows_v):
  c = lax.axis_index("c")
  s = lax.axis_index("s")
  wid = c * NS + s
  base_e = wid * ET
  r0 = s * ROWS_T

  pltpu.sync_copy(dinv_hbm, dinv_v)

  def zb(i, _):
    for j in range(8):
      zero_v[i, pl.ds(j * 16, 16)] = _zero_vec()
    return 0
  lax.fori_loop(0, CB, zb, 0)
  for kk in range(ROWS_T // CB):
    pltpu.sync_copy(zero_v, acc_sh.at[pl.ds(r0 + kk * CB, CB)])
  plsc.subcore_barrier()

  def scale_rows(nrows):
    def sb(j, _):
      fvv = fv[pl.ds(j * 16, 16)]
      for l in range(16):
        sv = fvv[l]
        row = rows_v.at[j * 16 + l]
        for q in range(8):
          row[pl.ds(q * 16, 16)] = row[pl.ds(q * 16, 16)] * sv
      return 0
    lax.fori_loop(0, nrows // 16, sb, 0)

  def chunk(k, _):
    off = base_e + k * CB
    pltpu.sync_copy(src_hbm.at[pl.ds(off, CB)], srcv)
    pltpu.sync_copy(dst_hbm.at[pl.ds(off, CB)], dstv)
    pltpu.sync_copy(ew_hbm.at[pl.ds(off, CB)], ewv)
    for j in range(CB // 16):
      sl = pl.ds(j * 16, 16)
      ds_ = plsc.load_gather(dinv_v, [srcv[sl]])
      dd_ = plsc.load_gather(dinv_v, [dstv[sl]])
      fv[sl] = ewv[sl] * ds_ * dd_
    pltpu.sync_copy(g_hbm.at[srcv], rows_v)
    scale_rows(CB)
    pltpu.sync_copy(rows_v, acc_sh.at[dstv], add=True)
    return 0
  lax.fori_loop(0, NCH, chunk, 0)

  sl_base = wid * SLT

  def slchunk(k, _):
    base = sl_base + k * CB
    for j in range(CB // 16):
      sl = pl.ds(j * 16, 16)
      ids_v[sl] = _iota16() + (base + j * 16)
      dv = dinv_v[pl.ds(base + j * 16, 16)]
      fv[sl] = dv * dv
    pltpu.sync_copy(g_hbm.at[pl.ds(base, CB)], rows_v)
    scale_rows(CB)
    pltpu.sync_copy(rows_v, acc_sh.at[ids_v], add=True)
    return 0
  lax.fori_loop(0, SLT // CB, slchunk, 0)

  plsc.subcore_barrier()
  if gather_idx:
    pltpu.sync_copy(idx_hbm.at[pl.ds(s * GIDX_T, GIDX_T)], idxg_v)
    pltpu.sync_copy(acc_sh.at[idxg_v], grows_v)
    pltpu.sync_copy(grows_v, rows_out.at[c, pl.ds(s * GIDX_T, GIDX_T)])
  else:
    pltpu.sync_copy(acc_sh.at[pl.ds(r0, ROWS_T)], acc_out.at[c, pl.ds(r0, ROWS_T)])


def _run_scatter(g, dinv, src, dst, ew, idx, gather_idx):
  scratch = [
      pltpu.VMEM_SHARED((NPAD, FEAT), jnp.float32),
      pltpu.VMEM((NPAD,), jnp.float32),
      pltpu.VMEM((CB,), jnp.int32),
      pltpu.VMEM((CB,), jnp.int32),
      pltpu.VMEM((CB,), jnp.float32),
      pltpu.VMEM((CB,), jnp.float32),
      pltpu.VMEM((CB, FEAT), jnp.float32),
      pltpu.VMEM((CB, FEAT), jnp.float32),
      pltpu.VMEM((CB,), jnp.int32),
      pltpu.VMEM((GIDX_T,), jnp.int32),
      pltpu.VMEM((GIDX_T, FEAT), jnp.float32),
  ]
  if gather_idx:
    out_type = [jax.ShapeDtypeStruct((NC, IDXN, FEAT), jnp.float32)]
    def body(g_, dinv_, src_, dst_, ew_, idx_, rows_out, *scr):
      _scatter_body(True, g_, dinv_, src_, dst_, ew_, idx_, None, rows_out, *scr)
  else:
    out_type = [jax.ShapeDtypeStruct((NC, NPAD, FEAT), jnp.float32)]
    def body(g_, dinv_, src_, dst_, ew_, idx_, acc_out, *scr):
      _scatter_body(False, g_, dinv_, src_, dst_, ew_, idx_, acc_out, None, *scr)
  k = pl.kernel(body, out_type=out_type, mesh=_mesh(), scratch_types=scratch,
                compiler_params=pltpu.CompilerParams(needs_layout_passes=False))
  return k(g, dinv, src, dst, ew, idx)[0]


def _tc1_body(vf, tf, degp, fc1wT, fc1b, fc2wT, fc2b, wAT, wBT, rb, c0wT,
              g0_ref, dinv_ref):
  f32 = jnp.float32
  v = jnp.dot(vf[...], fc1wT[...], preferred_element_type=f32) + fc1b[...]
  t = jnp.dot(tf[...], fc2wT[...], preferred_element_type=f32) + fc2b[...]
  h = (jnp.dot(v, wAT[...], preferred_element_type=f32)
       + jnp.dot(t, wBT[...], preferred_element_type=f32) + rb[...])
  h = jnp.where(h >= 0, h, 0.01 * h)
  g0_ref[...] = jnp.dot(h, c0wT[...], preferred_element_type=f32)
  deg = 1.0 + degp[0] + degp[1]
  dinv_ref[...] = lax.rsqrt(deg)


def _run_tc1(vf, tf, degp, fc1wT, fc1b, fc2wT, fc2b, wAT, wBT, rb, c0wT):
  BR = 640
  grid = (NPAD // BR,)
  full = lambda shape: pl.BlockSpec(shape, lambda i: tuple(0 for _ in shape))
  return pl.pallas_call(
      _tc1_body,
      grid=grid,
      in_specs=[
          pl.BlockSpec((BR, 16), lambda i: (i, 0)),
          pl.BlockSpec((BR, 768), lambda i: (i, 0)),
          full((NC, 80, 128)),
          full((16, 128)), full((1, 128)),
          full((768, 128)), full((1, 128)),
          full((128, 128)), full((128, 128)), full((1, 128)),
          full((128, 128)),
      ],
      out_specs=[
          pl.BlockSpec((BR, 128), lambda i: (i, 0)),
          pl.BlockSpec((80, 128), lambda i: (0, 0)),
      ],
      out_shape=[
          jax.ShapeDtypeStruct((NPAD, 128), jnp.float32),
          jax.ShapeDtypeStruct((80, 128), jnp.float32),
      ],
  )(vf, tf, degp, fc1wT, fc1b, fc2wT, fc2b, wAT, wBT, rb, c0wT)


def _tc2_body(acc, b0, c1wT, g1_ref):
  x = acc[0] + acc[1] + b0[...]
  g1_ref[...] = jnp.dot(x, c1wT[...], preferred_element_type=jnp.float32)


def _run_tc2(acc, b0, c1wT):
  BR = 640
  return pl.pallas_call(
      _tc2_body,
      grid=(NPAD // BR,),
      in_specs=[
          pl.BlockSpec((NC, BR, 128), lambda i: (0, i, 0)),
          pl.BlockSpec((1, 128), lambda i: (0, 0)),
          pl.BlockSpec((128, 128), lambda i: (0, 0)),
      ],
      out_specs=pl.BlockSpec((BR, 128), lambda i: (i, 0)),
      out_shape=jax.ShapeDtypeStruct((NPAD, 128), jnp.float32),
  )(acc, b0, c1wT)


def _tc3_body(rows, b1, fc3pT, fc3bp, out_ref):
  x = rows[0] + rows[1] + b1[...]
  out_ref[...] = jnp.dot(x, fc3pT[...], preferred_element_type=jnp.float32) + fc3bp[...]


def _run_tc3(rows, b1, fc3pT, fc3bp):
  return pl.pallas_call(
      _tc3_body,
      grid=(1,),
      in_specs=[
          pl.BlockSpec((NC, IDXN, 128), lambda i: (0, 0, 0)),
          pl.BlockSpec((1, 128), lambda i: (0, 0)),
          pl.BlockSpec((128, 128), lambda i: (0, 0)),
          pl.BlockSpec((1, 128), lambda i: (0, 0)),
      ],
      out_specs=pl.BlockSpec((IDXN, 128), lambda i: (0, 0)),
      out_shape=jax.ShapeDtypeStruct((IDXN, 128), jnp.float32),
  )(rows, b1, fc3pT, fc3bp)


def kernel(value_feature, text_feature, edge_index, edge_weight, idx,
           fc1_w, fc1_b, fc2_w, fc2_b, relu_w, relu_b,
           conv0_w, conv0_b, conv1_w, conv1_b, fc3_w, fc3_b):
  f32 = jnp.float32
  vf = jnp.zeros((NPAD, 16), f32).at[:N].set(value_feature)
  tf = jnp.zeros((NPAD, 768), f32).at[:N].set(text_feature)
  src = edge_index[0]
  dst = edge_index[1]

  fc1wT = fc1_w.T
  fc2wT = fc2_w.T
  wAT = relu_w[:, :128].T
  wBT = relu_w[:, 128:].T
  c0wT = conv0_w.T
  c1wT = conv1_w.T
  fc3pT = jnp.zeros((128, 128), f32).at[:, :2].set(fc3_w.T)
  fc3bp = jnp.zeros((1, 128), f32).at[0, :2].set(fc3_b)
  r1 = lambda b: b.reshape(1, 128)

  degp = _run_deg(dst, edge_weight)
  g0, dinv2d = _run_tc1(vf, tf, degp.reshape(NC, 80, 128), fc1wT, r1(fc1_b),
                        fc2wT, r1(fc2_b), wAT, wBT, r1(relu_b), c0wT)
  dinv = dinv2d.reshape(NPAD)
  acc0 = _run_scatter(g0, dinv, src, dst, edge_weight, idx, gather_idx=False)
  g1 = _run_tc2(acc0, r1(conv0_b), c1wT)
  rows = _run_scatter(g1, dinv, src, dst, edge_weight, idx, gather_idx=True)
  out = _run_tc3(rows, r1(conv1_b), fc3pT, fc3bp)
  return out[:, :2]


# speedup vs baseline: 14.3179x; 1.4928x over previous
import functools

import jax
import jax.numpy as jnp
from jax import lax
from jax.experimental import pallas as pl
from jax.experimental.pallas import tpu as pltpu
from jax.experimental.pallas import tpu_sc as plsc

N = 10000
NPAD = 10240
E = 320000
FEAT = 128
IDXN = 1024

NC = 2
NS = 16
NW = NC * NS
ET = E // NW
CB = 80
NCH = ET // CB
ROWS_T = NPAD // NS
SLT = NPAD // NW
GIDX_T = IDXN // NS

_mesh = lambda: plsc.VectorSubcoreMesh(core_axis_name="c", subcore_axis_name="s")


def _zero_vec():
  return jnp.zeros((16,), jnp.float32)


def _iota16():
  return lax.broadcasted_iota(jnp.int32, (16,), 0)


def _deg_kernel(dst_hbm, ew_hbm, out_hbm, deg_priv, dstv, ewv, slots_sh, red_v, red_o):
  c = lax.axis_index("c")
  s = lax.axis_index("s")
  wid = c * NS + s
  base_e = wid * ET

  def zbody(i, _):
    deg_priv[pl.ds(i * 16, 16)] = _zero_vec()
    return 0
  lax.fori_loop(0, NPAD // 16, zbody, 0)

  CE = 2000

  def chunk(k, _):
    off = base_e + k * CE
    pltpu.sync_copy(dst_hbm.at[pl.ds(off, CE)], dstv)
    pltpu.sync_copy(ew_hbm.at[pl.ds(off, CE)], ewv)

    def body(j, _):
      idx = dstv[pl.ds(j * 16, 16)]
      vals = ewv[pl.ds(j * 16, 16)]
      plsc.addupdate_scatter(deg_priv, [idx], vals)
      return 0
    lax.fori_loop(0, CE // 16, body, 0)
    return 0
  lax.fori_loop(0, ET // CE, chunk, 0)

  pltpu.sync_copy(deg_priv, slots_sh.at[s])
  plsc.subcore_barrier()
  part0 = s * (NPAD // NS)
  for t in range(NS):
    pltpu.sync_copy(slots_sh.at[t, pl.ds(part0, NPAD // NS)], red_v.at[t])

  def rbody(i, _):
    acc = red_v[0, pl.ds(i * 16, 16)]
    for t in range(1, NS):
      acc = acc + red_v[t, pl.ds(i * 16, 16)]
    red_o[pl.ds(i * 16, 16)] = acc
    return 0
  lax.fori_loop(0, (NPAD // NS) // 16, rbody, 0)
  pltpu.sync_copy(red_o, out_hbm.at[c, pl.ds(part0, NPAD // NS)])


def _run_deg(dst, ew):
  k = pl.kernel(
      _deg_kernel,
      out_type=jax.ShapeDtypeStruct((NC, NPAD), jnp.float32),
      mesh=_mesh(),
      compiler_params=pltpu.CompilerParams(needs_layout_passes=False),
      scratch_types=[
          pltpu.VMEM((NPAD,), jnp.float32),
          pltpu.VMEM((2000,), jnp.int32),
          pltpu.VMEM((2000,), jnp.float32),
          pltpu.VMEM_SHARED((NS, NPAD), jnp.float32),
          pltpu.VMEM((NS, NPAD // NS), jnp.float32),
          pltpu.VMEM((NPAD // NS,), jnp.float32),
      ],
  )
  return k(dst, ew)


NBUF = 3
NGRP = NCH // NBUF
NREM = NCH - NGRP * NBUF


def _scatter_body(gather_idx, g_hbm, dinv_hbm, src_hbm, dst_hbm, ew_hbm, idx_hbm,
                  acc_out, rows_out, acc_sh, dinv_v, src_st, dst_st, ew_st, fv_st,
                  rows, sems_g, sems_s, ids_v, idxg_v):
  c = lax.axis_index("c")
  s = lax.axis_index("s")
  wid = c * NS + s
  r0 = s * ROWS_T

  pltpu.sync_copy(dinv_hbm, dinv_v)

  sl_base = wid * SLT
  for k in range(SLT // CB):
    for j in range(CB // 16):
      ids_v[k, pl.ds(j * 16, 16)] = _iota16() + (sl_base + k * CB + j * 16)

  def zb(i, _):
    for j in range(8):
      rows[0][i, pl.ds(j * 16, 16)] = _zero_vec()
    return 0
  lax.fori_loop(0, CB, zb, 0)
  for kk in range(ROWS_T // CB):
    pltpu.sync_copy(rows[0], acc_sh.at[pl.ds(r0 + kk * CB, CB)])
  plsc.subcore_barrier()

  def stage(k0, nb):
    pltpu.sync_copy(src_hbm.at[wid, pl.ds(k0, nb)], src_st.at[pl.ds(0, nb)])
    pltpu.sync_copy(dst_hbm.at[wid, pl.ds(k0, nb)], dst_st.at[pl.ds(0, nb)])
    pltpu.sync_copy(ew_hbm.at[wid, pl.ds(k0, nb)], ew_st.at[pl.ds(0, nb)])
    for b in range(nb):
      for j in range(CB // 16):
        sl = pl.ds(j * 16, 16)
        ds_ = plsc.load_gather(dinv_v, [src_st[b, 0, sl]])
        dd_ = plsc.load_gather(dinv_v, [dst_st[b, 0, sl]])
        fv_st[b, sl] = ew_st[b, 0, sl] * ds_ * dd_

  def scale_rows(b, fb):
    def sb(j, _):
      fvv = fv_st[fb, pl.ds(j * 16, 16)]
      for l in range(16):
        sv = fvv[l]
        row = rows[b].at[j * 16 + l]
        for q in range(8):
          row[pl.ds(q * 16, 16)] = row[pl.ds(q * 16, 16)] * sv
      return 0
    lax.fori_loop(0, CB // 16, sb, 0)

  def outer(gi, _):
    k0 = gi * NBUF
    stage(k0, NBUF)
    gd = [pltpu.async_copy(g_hbm.at[src_st.at[b, 0]], rows[b], sems_g[b])
          for b in range(NBUF)]
    sd = []
    for b in range(NBUF):
      gd[b].wait()
      scale_rows(b, b)
      sd.append(pltpu.async_copy(rows[b], acc_sh.at[dst_st.at[b, 0]],
                                 sems_s[b], add=True))
    for d in sd:
      d.wait()
    return 0
  lax.fori_loop(0, NGRP, outer, 0)

  stage(NGRP * NBUF, NREM)
  for b in range(NREM):
    pltpu.sync_copy(g_hbm.at[src_st.at[b, 0]], rows[b])
    scale_rows(b, b)
    pltpu.sync_copy(rows[b], acc_sh.at[dst_st.at[b, 0]], add=True)

  for k in range(SLT // CB):
    base = sl_base + k * CB
    b = k % NBUF
    for j in range(CB // 16):
      sl = pl.ds(j * 16, 16)
      dv = dinv_v[pl.ds(base + j * 16, 16)]
      fv_st[0, sl] = dv * dv
    pltpu.sync_copy(g_hbm.at[pl.ds(base, CB)], rows[b])
    scale_rows(b, 0)
    pltpu.sync_copy(rows[b], acc_sh.at[ids_v.at[k]], add=True)

  plsc.subcore_barrier()
  if gather_idx:
    pltpu.sync_copy(idx_hbm.at[pl.ds(s * GIDX_T, GIDX_T)], idxg_v)
    pltpu.sync_copy(acc_sh.at[idxg_v], rows[0].at[pl.ds(0, GIDX_T)])
    pltpu.sync_copy(rows[0].at[pl.ds(0, GIDX_T)],
                    rows_out.at[c, pl.ds(s * GIDX_T, GIDX_T)])
  else:
    pltpu.sync_copy(acc_sh.at[pl.ds(r0, ROWS_T)], acc_out.at[c, pl.ds(r0, ROWS_T)])


def _run_scatter(g, dinv, src, dst, ew, idx, gather_idx):
  scratch = [
      pltpu.VMEM_SHARED((NPAD, FEAT), jnp.float32),
      pltpu.VMEM((NPAD,), jnp.float32),
      pltpu.VMEM((NBUF, 1, CB), jnp.int32),
      pltpu.VMEM((NBUF, 1, CB), jnp.int32),
      pltpu.VMEM((NBUF, 1, CB), jnp.float32),
      pltpu.VMEM((NBUF, CB), jnp.float32),
      [pltpu.VMEM((CB, FEAT), jnp.float32) for _ in range(NBUF)],
      [pltpu.SemaphoreType.DMA for _ in range(NBUF)],
      [pltpu.SemaphoreType.DMA for _ in range(NBUF)],
      pltpu.VMEM((SLT // CB, CB), jnp.int32),
      pltpu.VMEM((GIDX_T,), jnp.int32),
  ]
  if gather_idx:
    out_type = [jax.ShapeDtypeStruct((NC, IDXN, FEAT), jnp.float32)]
    def body(g_, dinv_, src_, dst_, ew_, idx_, rows_out, *scr):
      _scatter_body(True, g_, dinv_, src_, dst_, ew_, idx_, None, rows_out, *scr)
  else:
    out_type = [jax.ShapeDtypeStruct((NC, NPAD, FEAT), jnp.float32)]
    def body(g_, dinv_, src_, dst_, ew_, idx_, acc_out, *scr):
      _scatter_body(False, g_, dinv_, src_, dst_, ew_, idx_, acc_out, None, *scr)
  k = pl.kernel(body, out_type=out_type, mesh=_mesh(), scratch_types=scratch,
                compiler_params=pltpu.CompilerParams(needs_layout_passes=False))
  src3 = src.reshape(NW, NCH, 1, CB)
  dst3 = dst.reshape(NW, NCH, 1, CB)
  ew3 = ew.reshape(NW, NCH, 1, CB)
  return k(g, dinv, src3, dst3, ew3, idx)[0]


def _tc1_body(vf, tf, degp, fc1wT, fc1b, fc2wT, fc2b, wAT, wBT, rb, c0wT,
              g0_ref, dinv_ref):
  f32 = jnp.float32
  v = jnp.dot(vf[...], fc1wT[...], preferred_element_type=f32) + fc1b[...]
  t = jnp.dot(tf[...], fc2wT[...], preferred_element_type=f32) + fc2b[...]
  h = (jnp.dot(v, wAT[...], preferred_element_type=f32)
       + jnp.dot(t, wBT[...], preferred_element_type=f32) + rb[...])
  h = jnp.where(h >= 0, h, 0.01 * h)
  g0_ref[...] = jnp.dot(h, c0wT[...], preferred_element_type=f32)
  deg = 1.0 + degp[0] + degp[1]
  dinv_ref[...] = lax.rsqrt(deg)


def _run_tc1(vf, tf, degp, fc1wT, fc1b, fc2wT, fc2b, wAT, wBT, rb, c0wT):
  BR = 640
  grid = (NPAD // BR,)
  full = lambda shape: pl.BlockSpec(shape, lambda i: tuple(0 for _ in shape))
  return pl.pallas_call(
      _tc1_body,
      grid=grid,
      in_specs=[
          pl.BlockSpec((BR, 16), lambda i: (i, 0)),
          pl.BlockSpec((BR, 768), lambda i: (i, 0)),
          full((NC, 80, 128)),
          full((16, 128)), full((1, 128)),
          full((768, 128)), full((1, 128)),
          full((128, 128)), full((128, 128)), full((1, 128)),
          full((128, 128)),
      ],
      out_specs=[
          pl.BlockSpec((BR, 128), lambda i: (i, 0)),
          pl.BlockSpec((80, 128), lambda i: (0, 0)),
      ],
      out_shape=[
          jax.ShapeDtypeStruct((NPAD, 128), jnp.float32),
          jax.ShapeDtypeStruct((80, 128), jnp.float32),
      ],
  )(vf, tf, degp, fc1wT, fc1b, fc2wT, fc2b, wAT, wBT, rb, c0wT)


def _tc2_body(acc, b0, c1wT, g1_ref):
  x = acc[0] + acc[1] + b0[...]
  g1_ref[...] = jnp.dot(x, c1wT[...], preferred_element_type=jnp.float32)


def _run_tc2(acc, b0, c1wT):
  BR = 640
  return pl.pallas_call(
      _tc2_body,
      grid=(NPAD // BR,),
      in_specs=[
          pl.BlockSpec((NC, BR, 128), lambda i: (0, i, 0)),
          pl.BlockSpec((1, 128), lambda i: (0, 0)),
          pl.BlockSpec((128, 128), lambda i: (0, 0)),
      ],
      out_specs=pl.BlockSpec((BR, 128), lambda i: (i, 0)),
      out_shape=jax.ShapeDtypeStruct((NPAD, 128), jnp.float32),
  )(acc, b0, c1wT)


def _tc3_body(rows, b1, fc3pT, fc3bp, out_ref):
  x = rows[0] + rows[1] + b1[...]
  out_ref[...] = jnp.dot(x, fc3pT[...], preferred_element_type=jnp.float32) + fc3bp[...]


def _run_tc3(rows, b1, fc3pT, fc3bp):
  return pl.pallas_call(
      _tc3_body,
      grid=(1,),
      in_specs=[
          pl.BlockSpec((NC, IDXN, 128), lambda i: (0, 0, 0)),
          pl.BlockSpec((1, 128), lambda i: (0, 0)),
          pl.BlockSpec((128, 128), lambda i: (0, 0)),
          pl.BlockSpec((1, 128), lambda i: (0, 0)),
      ],
      out_specs=pl.BlockSpec((IDXN, 128), lambda i: (0, 0)),
      out_shape=jax.ShapeDtypeStruct((IDXN, 128), jnp.float32),
  )(rows, b1, fc3pT, fc3bp)


def kernel(value_feature, text_feature, edge_index, edge_weight, idx,
           fc1_w, fc1_b, fc2_w, fc2_b, relu_w, relu_b,
           conv0_w, conv0_b, conv1_w, conv1_b, fc3_w, fc3_b):
  f32 = jnp.float32
  vf = jnp.zeros((NPAD, 16), f32).at[:N].set(value_feature)
  tf = jnp.zeros((NPAD, 768), f32).at[:N].set(text_feature)
  src = edge_index[0]
  dst = edge_index[1]

  fc1wT = fc1_w.T
  fc2wT = fc2_w.T
  wAT = relu_w[:, :128].T
  wBT = relu_w[:, 128:].T
  c0wT = conv0_w.T
  c1wT = conv1_w.T
  fc3pT = jnp.zeros((128, 128), f32).at[:, :2].set(fc3_w.T)
  fc3bp = jnp.zeros((1, 128), f32).at[0, :2].set(fc3_b)
  r1 = lambda b: b.reshape(1, 128)

  degp = _run_deg(dst, edge_weight)
  g0, dinv2d = _run_tc1(vf, tf, degp.reshape(NC, 80, 128), fc1wT, r1(fc1_b),
                        fc2wT, r1(fc2_b), wAT, wBT, r1(relu_b), c0wT)
  dinv = dinv2d.reshape(NPAD)
  acc0 = _run_scatter(g0, dinv, src, dst, edge_weight, idx, gather_idx=False)
  g1 = _run_tc2(acc0, r1(conv0_b), c1wT)
  rows = _run_scatter(g1, dinv, src, dst, edge_weight, idx, gather_idx=True)
  out = _run_tc3(rows, r1(conv1_b), fc3pT, fc3bp)
  return out[:, :2]


# speedup vs baseline: 15.9685x; 1.1153x over previous
import functools

import jax
import jax.numpy as jnp
from jax import lax
from jax.experimental import pallas as pl
from jax.experimental.pallas import tpu as pltpu
from jax.experimental.pallas import tpu_sc as plsc

N = 10000
NPAD = 10240
E = 320000
FEAT = 128
IDXN = 1024

NC = 2
NS = 16
NW = NC * NS
ET = E // NW
CB = 80
NCH = ET // CB
ROWS_T = NPAD // NS
SLT = NPAD // NW
GIDX_T = IDXN // NS

_mesh = lambda: plsc.VectorSubcoreMesh(core_axis_name="c", subcore_axis_name="s")


def _zero_vec():
  return jnp.zeros((16,), jnp.float32)


def _iota16():
  return lax.broadcasted_iota(jnp.int32, (16,), 0)


def _deg_kernel(dst_hbm, ew_hbm, out_hbm, deg_priv, dstv, ewv, slots_sh, red_v, red_o):
  c = lax.axis_index("c")
  s = lax.axis_index("s")
  wid = c * NS + s
  base_e = wid * ET

  def zbody(i, _):
    deg_priv[pl.ds(i * 16, 16)] = _zero_vec()
    return 0
  lax.fori_loop(0, NPAD // 16, zbody, 0)

  CE = 2000

  def chunk(k, _):
    off = base_e + k * CE
    pltpu.sync_copy(dst_hbm.at[pl.ds(off, CE)], dstv)
    pltpu.sync_copy(ew_hbm.at[pl.ds(off, CE)], ewv)

    def body(j, _):
      idx = dstv[pl.ds(j * 16, 16)]
      vals = ewv[pl.ds(j * 16, 16)]
      plsc.addupdate_scatter(deg_priv, [idx], vals)
      return 0
    lax.fori_loop(0, CE // 16, body, 0)
    return 0
  lax.fori_loop(0, ET // CE, chunk, 0)

  pltpu.sync_copy(deg_priv, slots_sh.at[s])
  plsc.subcore_barrier()
  part0 = s * (NPAD // NS)
  for t in range(NS):
    pltpu.sync_copy(slots_sh.at[t, pl.ds(part0, NPAD // NS)], red_v.at[t])

  def rbody(i, _):
    acc = red_v[0, pl.ds(i * 16, 16)]
    for t in range(1, NS):
      acc = acc + red_v[t, pl.ds(i * 16, 16)]
    red_o[pl.ds(i * 16, 16)] = acc
    return 0
  lax.fori_loop(0, (NPAD // NS) // 16, rbody, 0)
  pltpu.sync_copy(red_o, out_hbm.at[c, pl.ds(part0, NPAD // NS)])


def _run_deg(dst, ew):
  k = pl.kernel(
      _deg_kernel,
      out_type=jax.ShapeDtypeStruct((NC, NPAD), jnp.float32),
      mesh=_mesh(),
      compiler_params=pltpu.CompilerParams(needs_layout_passes=False),
      scratch_types=[
          pltpu.VMEM((NPAD,), jnp.float32),
          pltpu.VMEM((2000,), jnp.int32),
          pltpu.VMEM((2000,), jnp.float32),
          pltpu.VMEM_SHARED((NS, NPAD), jnp.float32),
          pltpu.VMEM((NS, NPAD // NS), jnp.float32),
          pltpu.VMEM((NPAD // NS,), jnp.float32),
      ],
  )
  return k(dst, ew)


NBUF = 4
GROUP = NBUF
NGRP_MAIN = 30
NTAIL = NCH - NGRP_MAIN * GROUP


def _scatter_body(gather_idx, g_hbm, dinv_hbm, src_hbm, dst_hbm, ew_hbm, idx_hbm,
                  acc_out, rows_out, acc_sh, src_st, dst_st, ew_st, dsv, ddv,
                  rows, st_sems, gsem, dsem, ddsem, ssem, ids_v, idxg_v):
  c = lax.axis_index("c")
  s = lax.axis_index("s")
  wid = c * NS + s
  r0 = s * ROWS_T

  sl_base = wid * SLT
  for k in range(SLT // CB):
    for j in range(CB // 16):
      ids_v[k, pl.ds(j * 16, 16)] = _iota16() + (sl_base + k * CB + j * 16)

  def zb(i, _):
    for j in range(8):
      rows[0][i, pl.ds(j * 16, 16)] = _zero_vec()
    return 0
  lax.fori_loop(0, CB, zb, 0)
  for kk in range(ROWS_T // CB):
    pltpu.sync_copy(rows[0], acc_sh.at[pl.ds(r0 + kk * CB, CB)])
  plsc.subcore_barrier()

  def stage_refs(p, g):
    k0 = g * GROUP
    return ((src_hbm.at[wid, pl.ds(k0, GROUP)], src_st.at[p], st_sems[p][0]),
            (dst_hbm.at[wid, pl.ds(k0, GROUP)], dst_st.at[p], st_sems[p][1]),
            (ew_hbm.at[wid, pl.ds(k0, GROUP)], ew_st.at[p], st_sems[p][2]))

  def stage_issue(p, g):
    for a, b_, sm in stage_refs(p, g):
      pltpu.async_copy(a, b_, sm)

  def stage_wait(p, g):
    for a, b_, sm in stage_refs(p, g):
      pltpu.make_async_copy(a, b_, sm).wait()

  def scale(mk_f, b):
    def sb(j, _):
      fvv = mk_f(b, pl.ds(j * 16, 16))
      for l in range(16):
        sv = fvv[l]
        row = rows[b].at[j * 16 + l]
        for q in range(8):
          row[pl.ds(q * 16, 16)] = row[pl.ds(q * 16, 16)] * sv
      return 0
    lax.fori_loop(0, CB // 16, sb, 0)

  def edge_f(p):
    def mk_f(b, sl):
      return ew_st[p, b, 0, sl] * dsv[b, sl] * ddv[b, sl]
    return mk_f

  def half_step(p, g):
    stage_wait(p, g)
    dgd = []
    for b in range(GROUP):
      dgd.append(pltpu.async_copy(dinv_hbm.at[src_st.at[p, b, 0]], dsv.at[b],
                                  dsem[b]))
      dgd.append(pltpu.async_copy(dinv_hbm.at[dst_st.at[p, b, 0]], ddv.at[b],
                                  ddsem[b]))
    gd = [pltpu.async_copy(g_hbm.at[src_st.at[p, b, 0]], rows[b], gsem[b])
          for b in range(GROUP)]
    sd = []
    for b in range(GROUP):
      gd[b].wait()
      dgd[2 * b].wait()
      dgd[2 * b + 1].wait()
      scale(edge_f(p), b)
      sd.append(pltpu.async_copy(rows[b], acc_sh.at[dst_st.at[p, b, 0]],
                                 ssem[b], add=True))
    for d in sd:
      d.wait()
    @pl.when(g + 2 < NGRP_MAIN)
    def _():
      stage_issue(p, g + 2)

  stage_issue(0, 0)
  stage_issue(1, 1)

  def outer(i, _):
    half_step(0, 2 * i)
    half_step(1, 2 * i + 1)
    return 0
  lax.fori_loop(0, NGRP_MAIN // 2, outer, 0)

  def tail(t, _):
    k = NGRP_MAIN * GROUP + t
    pltpu.sync_copy(src_hbm.at[wid, pl.ds(k, 1)], src_st.at[0, pl.ds(0, 1)])
    pltpu.sync_copy(dst_hbm.at[wid, pl.ds(k, 1)], dst_st.at[0, pl.ds(0, 1)])
    pltpu.sync_copy(ew_hbm.at[wid, pl.ds(k, 1)], ew_st.at[0, pl.ds(0, 1)])
    pltpu.sync_copy(dinv_hbm.at[src_st.at[0, 0, 0]], dsv.at[0])
    pltpu.sync_copy(dinv_hbm.at[dst_st.at[0, 0, 0]], ddv.at[0])
    pltpu.sync_copy(g_hbm.at[src_st.at[0, 0, 0]], rows[0])
    scale(edge_f(0), 0)
    pltpu.sync_copy(rows[0], acc_sh.at[dst_st.at[0, 0, 0]], add=True)
    return 0
  lax.fori_loop(0, NTAIL, tail, 0)

  def slbody(k, _):
    base = sl_base + k * CB
    pltpu.sync_copy(dinv_hbm.at[pl.ds(base, CB)], dsv.at[0])
    pltpu.sync_copy(g_hbm.at[pl.ds(base, CB)], rows[0])
    def mk_f(b, sl):
      dv = dsv[0, sl]
      return dv * dv
    scale(mk_f, 0)
    pltpu.sync_copy(rows[0], acc_sh.at[ids_v.at[k]], add=True)
    return 0
  lax.fori_loop(0, SLT // CB, slbody, 0)

  plsc.subcore_barrier()
  if gather_idx:
    pltpu.sync_copy(idx_hbm.at[pl.ds(s * GIDX_T, GIDX_T)], idxg_v)
    pltpu.sync_copy(acc_sh.at[idxg_v], rows[0].at[pl.ds(0, GIDX_T)])
    pltpu.sync_copy(rows[0].at[pl.ds(0, GIDX_T)],
                    rows_out.at[c, pl.ds(s * GIDX_T, GIDX_T)])
  else:
    pltpu.sync_copy(acc_sh.at[pl.ds(r0, ROWS_T)], acc_out.at[c, pl.ds(r0, ROWS_T)])


def _run_scatter(g, dinv, src, dst, ew, idx, gather_idx):
  scratch = [
      pltpu.VMEM_SHARED((NPAD, FEAT), jnp.float32),
      pltpu.VMEM((2, GROUP, 1, CB), jnp.int32),
      pltpu.VMEM((2, GROUP, 1, CB), jnp.int32),
      pltpu.VMEM((2, GROUP, 1, CB), jnp.float32),
      pltpu.VMEM((GROUP, CB), jnp.float32),
      pltpu.VMEM((GROUP, CB), jnp.float32),
      [pltpu.VMEM((CB, FEAT), jnp.float32) for _ in range(NBUF)],
      [[pltpu.SemaphoreType.DMA for _ in range(3)] for _ in range(2)],
      [pltpu.SemaphoreType.DMA for _ in range(NBUF)],
      [pltpu.SemaphoreType.DMA for _ in range(NBUF)],
      [pltpu.SemaphoreType.DMA for _ in range(NBUF)],
      [pltpu.SemaphoreType.DMA for _ in range(NBUF)],
      pltpu.VMEM((SLT // CB, CB), jnp.int32),
      pltpu.VMEM((GIDX_T,), jnp.int32),
  ]
  if gather_idx:
    out_type = [jax.ShapeDtypeStruct((NC, IDXN, FEAT), jnp.float32)]
    def body(g_, dinv_, src_, dst_, ew_, idx_, rows_out, *scr):
      _scatter_body(True, g_, dinv_, src_, dst_, ew_, idx_, None, rows_out, *scr)
  else:
    out_type = [jax.ShapeDtypeStruct((NC, NPAD, FEAT), jnp.float32)]
    def body(g_, dinv_, src_, dst_, ew_, idx_, acc_out, *scr):
      _scatter_body(False, g_, dinv_, src_, dst_, ew_, idx_, acc_out, None, *scr)
  k = pl.kernel(body, out_type=out_type, mesh=_mesh(), scratch_types=scratch,
                compiler_params=pltpu.CompilerParams(needs_layout_passes=False))
  src3 = src.reshape(NW, NCH, 1, CB)
  dst3 = dst.reshape(NW, NCH, 1, CB)
  ew3 = ew.reshape(NW, NCH, 1, CB)
  return k(g, dinv, src3, dst3, ew3, idx)[0]


def _tc1_body(vf, tf, degp, fc1wT, fc1b, fc2wT, fc2b, wAT, wBT, rb, c0wT,
              g0_ref, dinv_ref):
  f32 = jnp.float32
  v = jnp.dot(vf[...], fc1wT[...], preferred_element_type=f32) + fc1b[...]
  t = jnp.dot(tf[...], fc2wT[...], preferred_element_type=f32) + fc2b[...]
  h = (jnp.dot(v, wAT[...], preferred_element_type=f32)
       + jnp.dot(t, wBT[...], preferred_element_type=f32) + rb[...])
  h = jnp.where(h >= 0, h, 0.01 * h)
  g0_ref[...] = jnp.dot(h, c0wT[...], preferred_element_type=f32)
  deg = 1.0 + degp[0] + degp[1]
  dinv_ref[...] = lax.rsqrt(deg)


def _run_tc1(vf, tf, degp, fc1wT, fc1b, fc2wT, fc2b, wAT, wBT, rb, c0wT):
  BR = 640
  grid = (NPAD // BR,)
  full = lambda shape: pl.BlockSpec(shape, lambda i: tuple(0 for _ in shape))
  return pl.pallas_call(
      _tc1_body,
      grid=grid,
      in_specs=[
          pl.BlockSpec((BR, 16), lambda i: (i, 0)),
          pl.BlockSpec((BR, 768), lambda i: (i, 0)),
          full((NC, 80, 128)),
          full((16, 128)), full((1, 128)),
          full((768, 128)), full((1, 128)),
          full((128, 128)), full((128, 128)), full((1, 128)),
          full((128, 128)),
      ],
      out_specs=[
          pl.BlockSpec((BR, 128), lambda i: (i, 0)),
          pl.BlockSpec((80, 128), lambda i: (0, 0)),
      ],
      out_shape=[
          jax.ShapeDtypeStruct((NPAD, 128), jnp.float32),
          jax.ShapeDtypeStruct((80, 128), jnp.float32),
      ],
  )(vf, tf, degp, fc1wT, fc1b, fc2wT, fc2b, wAT, wBT, rb, c0wT)


def _tc2_body(acc, b0, c1wT, g1_ref):
  x = acc[0] + acc[1] + b0[...]
  g1_ref[...] = jnp.dot(x, c1wT[...], preferred_element_type=jnp.float32)


def _run_tc2(acc, b0, c1wT):
  BR = 640
  return pl.pallas_call(
      _tc2_body,
      grid=(NPAD // BR,),
      in_specs=[
          pl.BlockSpec((NC, BR, 128), lambda i: (0, i, 0)),
          pl.BlockSpec((1, 128), lambda i: (0, 0)),
          pl.BlockSpec((128, 128), lambda i: (0, 0)),
      ],
      out_specs=pl.BlockSpec((BR, 128), lambda i: (i, 0)),
      out_shape=jax.ShapeDtypeStruct((NPAD, 128), jnp.float32),
  )(acc, b0, c1wT)


def _tc3_body(rows, b1, fc3pT, fc3bp, out_ref):
  x = rows[0] + rows[1] + b1[...]
  out_ref[...] = jnp.dot(x, fc3pT[...], preferred_element_type=jnp.float32) + fc3bp[...]


def _run_tc3(rows, b1, fc3pT, fc3bp):
  return pl.pallas_call(
      _tc3_body,
      grid=(1,),
      in_specs=[
          pl.BlockSpec((NC, IDXN, 128), lambda i: (0, 0, 0)),
          pl.BlockSpec((1, 128), lambda i: (0, 0)),
          pl.BlockSpec((128, 128), lambda i: (0, 0)),
          pl.BlockSpec((1, 128), lambda i: (0, 0)),
      ],
      out_specs=pl.BlockSpec((IDXN, 128), lambda i: (0, 0)),
      out_shape=jax.ShapeDtypeStruct((IDXN, 128), jnp.float32),
  )(rows, b1, fc3pT, fc3bp)


def kernel(value_feature, text_feature, edge_index, edge_weight, idx,
           fc1_w, fc1_b, fc2_w, fc2_b, relu_w, relu_b,
           conv0_w, conv0_b, conv1_w, conv1_b, fc3_w, fc3_b):
  f32 = jnp.float32
  vf = jnp.zeros((NPAD, 16), f32).at[:N].set(value_feature)
  tf = jnp.zeros((NPAD, 768), f32).at[:N].set(text_feature)
  src = edge_index[0]
  dst = edge_index[1]

  fc1wT = fc1_w.T
  fc2wT = fc2_w.T
  wAT = relu_w[:, :128].T
  wBT = relu_w[:, 128:].T
  c0wT = conv0_w.T
  c1wT = conv1_w.T
  fc3pT = jnp.zeros((128, 128), f32).at[:, :2].set(fc3_w.T)
  fc3bp = jnp.zeros((1, 128), f32).at[0, :2].set(fc3_b)
  r1 = lambda b: b.reshape(1, 128)

  degp = _run_deg(dst, edge_weight)
  g0, dinv2d = _run_tc1(vf, tf, degp.reshape(NC, 80, 128), fc1wT, r1(fc1_b),
                        fc2wT, r1(fc2_b), wAT, wBT, r1(relu_b), c0wT)
  dinv = dinv2d.reshape(NPAD)
  acc0 = _run_scatter(g0, dinv, src, dst, edge_weight, idx, gather_idx=False)
  g1 = _run_tc2(acc0, r1(conv0_b), c1wT)
  rows = _run_scatter(g1, dinv, src, dst, edge_weight, idx, gather_idx=True)
  out = _run_tc3(rows, r1(conv1_b), fc3pT, fc3bp)
  return out[:, :2]
